# Initial kernel scaffold; baseline (speedup 1.0000x reference)
import jax
import jax.numpy as jnp
from jax.experimental import pallas as pl


def kernel(x, edge_index, W1, b1, W2, b2):
    raise NotImplementedError("write your pallas kernel here")


# speedup vs baseline: 25.7021x; 25.7021x over previous
import functools

import jax
import jax.numpy as jnp
from jax import lax
from jax.experimental import pallas as pl
from jax.experimental.pallas import tpu as pltpu
from jax.experimental.pallas import tpu_sc as plsc

N = 10000
D = 128
E = 320000

NC = 2
NS = 16
CH = 128
EP = 327680
NCHT = EP // (NC * NS * CH)
NPAD = 10112
RPT = NPAD // NS

_mesh = plsc.VectorSubcoreMesh(core_axis_name="c", subcore_axis_name="s")


@functools.partial(
    pl.kernel,
    out_type=jax.ShapeDtypeStruct((NC, NS, RPT, D), jnp.float32),
    mesh=_mesh,
    scratch_types=[
        pltpu.VMEM((NCHT, CH), jnp.int32),
        pltpu.VMEM((CH, D), jnp.float32),
        pltpu.VMEM_SHARED((NPAD, D), jnp.float32),
        pltpu.SemaphoreType.DMA,
    ],
)
def _sc_counts(col_hbm, ones_hbm, zer_hbm, out_hbm, idx_v, ones_v, csh, sem):
    c = lax.axis_index("c")
    s = lax.axis_index("s")
    pltpu.sync_copy(zer_hbm, csh.at[pl.ds(s * RPT, RPT)])
    pltpu.sync_copy(ones_hbm, ones_v)
    pltpu.sync_copy(col_hbm.at[pl.ds(c * (NS * NCHT) + s * NCHT, NCHT)], idx_v)
    plsc.subcore_barrier()

    def body(j, carry):
        for k in range(8):
            pltpu.async_copy(ones_v, csh.at[idx_v.at[8 * j + k]], sem, add=True)
        for k in range(8):
            pltpu.make_async_copy(ones_v, csh.at[idx_v.at[8 * j + k]], sem).wait()
        return carry

    lax.fori_loop(0, NCHT // 8, body, 0)
    plsc.subcore_barrier()
    pltpu.sync_copy(csh.at[pl.ds(s * RPT, RPT)], out_hbm.at[c, s])


@functools.partial(
    pl.kernel,
    out_type=jax.ShapeDtypeStruct((NC, NS, RPT, D), jnp.float32),
    mesh=_mesh,
    scratch_types=[
        pltpu.VMEM((NCHT // 2, CH), jnp.int32),
        pltpu.VMEM((NCHT // 2, CH), jnp.int32),
        pltpu.VMEM((CH, D), jnp.float32),
        pltpu.VMEM((CH, D), jnp.float32),
        pltpu.VMEM_SHARED((NPAD, D), jnp.float32),
        pltpu.SemaphoreType.DMA,
        pltpu.SemaphoreType.DMA,
        pltpu.SemaphoreType.DMA,
        pltpu.SemaphoreType.DMA,
    ],
)
def _sc_scatter(g_hbm, row_hbm, col_hbm, zer_hbm, out_hbm,
                idxr_v, idxc_v, buf0, buf1, ssh, sg0, sg1, ss0, ss1):
    c = lax.axis_index("c")
    s = lax.axis_index("s")
    pltpu.sync_copy(zer_hbm, ssh.at[pl.ds(s * RPT, RPT)])
    base = c * (NS * NCHT) + s * NCHT
    plsc.subcore_barrier()

    def gissue(idx_j, buf, sem):
        for h in range(2):
            pltpu.async_copy(g_hbm.at[idxr_v.at[idx_j, pl.ds(64 * h, 64)]],
                             buf.at[pl.ds(64 * h, 64)], sem)

    def gwait(idx_j, buf, sem):
        for h in range(2):
            pltpu.make_async_copy(g_hbm.at[idxr_v.at[idx_j, pl.ds(64 * h, 64)]],
                                  buf.at[pl.ds(64 * h, 64)], sem).wait()

    HALF = NCHT // 2
    for p in range(2):
        pltpu.sync_copy(row_hbm.at[pl.ds(base + p * HALF, HALF)], idxr_v)
        pltpu.sync_copy(col_hbm.at[pl.ds(base + p * HALF, HALF)], idxc_v)
        gissue(0, buf0, sg0)

        def body(j, carry):
            gissue(2 * j + 1, buf1, sg1)
            gwait(2 * j, buf0, sg0)
            pltpu.sync_copy(buf0, ssh.at[idxc_v.at[2 * j]], add=True)
            gissue(jnp.minimum(2 * j + 2, HALF - 1), buf0, sg0)
            gwait(2 * j + 1, buf1, sg1)
            pltpu.sync_copy(buf1, ssh.at[idxc_v.at[2 * j + 1]], add=True)
            return carry

        lax.fori_loop(0, HALF // 2, body, 0)
        gwait(HALF - 1, buf0, sg0)
    plsc.subcore_barrier()
    pltpu.sync_copy(ssh.at[pl.ds(s * RPT, RPT)], out_hbm.at[c, s])


_RB = RPT
_GRID = NPAD // _RB


def _dinv_of(c_ref):
    deg = 1.0 + c_ref[0, :, 0:1] + c_ref[1, :, 0:1]
    return lax.rsqrt(deg)


def _mm1_body(x_ref, w_ref, c_ref, o_ref):
    h = jnp.dot(x_ref[...], w_ref[...], preferred_element_type=jnp.float32)
    o_ref[...] = h * _dinv_of(c_ref)


def _mm2_body(s_ref, g_ref, c_ref, w_ref, b_ref, o_ref):
    dinv = _dinv_of(c_ref)
    t = (s_ref[0] + s_ref[1] + g_ref[...]) * dinv + b_ref[...]
    h = jnp.maximum(t, 0.0)
    o_ref[...] = jnp.dot(h, w_ref[...], preferred_element_type=jnp.float32) * dinv


def _fin_body(s_ref, g_ref, c_ref, b_ref, o_ref):
    o_ref[...] = (s_ref[0] + s_ref[1] + g_ref[...]) * _dinv_of(c_ref) + b_ref[...]


_CBLK = pl.BlockSpec((NC, _RB, D), lambda i: (0, i, 0))

_mm1_tc = pl.pallas_call(
    _mm1_body,
    grid=(_GRID,),
    in_specs=[
        pl.BlockSpec((_RB, D), lambda i: (i, 0)),
        pl.BlockSpec((D, D), lambda i: (0, 0)),
        _CBLK,
    ],
    out_specs=pl.BlockSpec((_RB, D), lambda i: (i, 0)),
    out_shape=jax.ShapeDtypeStruct((NPAD, D), jnp.float32),
)

_mm2_tc = pl.pallas_call(
    _mm2_body,
    grid=(_GRID,),
    in_specs=[
        pl.BlockSpec((NC, _RB, D), lambda i: (0, i, 0)),
        pl.BlockSpec((_RB, D), lambda i: (i, 0)),
        _CBLK,
        pl.BlockSpec((D, D), lambda i: (0, 0)),
        pl.BlockSpec((1, D), lambda i: (0, 0)),
    ],
    out_specs=pl.BlockSpec((_RB, D), lambda i: (i, 0)),
    out_shape=jax.ShapeDtypeStruct((NPAD, D), jnp.float32),
)

_fin_tc = pl.pallas_call(
    _fin_body,
    grid=(_GRID,),
    in_specs=[
        pl.BlockSpec((NC, _RB, D), lambda i: (0, i, 0)),
        pl.BlockSpec((_RB, D), lambda i: (i, 0)),
        _CBLK,
        pl.BlockSpec((1, D), lambda i: (0, 0)),
    ],
    out_specs=pl.BlockSpec((_RB, D), lambda i: (i, 0)),
    out_shape=jax.ShapeDtypeStruct((NPAD, D), jnp.float32),
)


def kernel(x, edge_index, W1, b1, W2, b2):
    row = edge_index[0]
    col = edge_index[1]
    npad = EP - E
    pad_src = (jnp.arange(npad, dtype=jnp.int32) * 61) % N
    pad_dst = N + (jnp.arange(npad, dtype=jnp.int32) % (NPAD - N))
    rowp = jnp.concatenate([row, pad_src]).reshape(EP // CH, CH)
    colp = jnp.concatenate([col, pad_dst]).reshape(EP // CH, CH)

    ones128 = jnp.ones((CH, D), jnp.float32)
    zer128 = jnp.zeros((RPT, D), jnp.float32)
    b1r = b1.reshape(1, D)
    b2r = b2.reshape(1, D)

    xp = jnp.concatenate([x, jnp.zeros((NPAD - N, D), jnp.float32)], axis=0)
    counts = _sc_counts(colp, ones128, zer128).reshape(NC, NPAD, D)
    g1 = _mm1_tc(xp, W1, counts)
    s1 = _sc_scatter(g1, rowp, colp, zer128).reshape(NC, NPAD, D)
    g2 = _mm2_tc(s1, g1, counts, W2, b1r)
    s2 = _sc_scatter(g2, rowp, colp, zer128).reshape(NC, NPAD, D)
    return _fin_tc(s2, g2, counts, b2r)[:N]
